# baseline (device time: 84847 ns/iter reference)
import jax
import jax.numpy as jnp
from jax import lax
from jax.experimental import pallas as pl
from jax.experimental.pallas import tpu as pltpu

N_DEV = 16
LOG2_N = 4
B, SQ, SKV, DH = 2, 256, 256, 64
H_LOC = 4
D_MODEL = 512
HD_LOC = H_LOC * DH


def kernel(x, Wq, K_ext, V_ext, Wo):
    pos = lax.axis_index("i")
    Wq_loc = lax.dynamic_slice(Wq, (0, pos * HD_LOC), (D_MODEL, HD_LOC))
    Wo_loc = lax.dynamic_slice(Wo, (pos * HD_LOC, 0), (HD_LOC, D_MODEL))

    x_bf = x.astype(jnp.bfloat16)
    wq_bf = Wq_loc.astype(jnp.bfloat16)
    k_bf = K_ext.astype(jnp.bfloat16)
    v_bf = V_ext.astype(jnp.bfloat16)
    wo_bf = Wo_loc.astype(jnp.bfloat16)

    def body(x_ref, wq_ref, k_ref, v_ref, wo_ref, out_ref,
             send_buf, recv_buf, send_sems, recv_sems):
        my = lax.axis_index("i")

        rb = lax.broadcasted_iota(jnp.int32, (SQ, SKV), 0) // 64
        cb = lax.broadcasted_iota(jnp.int32, (SQ, SKV), 1) // 64
        mask = (rb == cb) | ((cb % 4) == (rb % 4))

        for b in range(B):
            q = jnp.dot(x_ref[b], wq_ref[...],
                        preferred_element_type=jnp.float32)
            q = q.astype(jnp.bfloat16)
            ctxs = []
            for h in range(H_LOC):
                qh = q[:, h * DH:(h + 1) * DH]
                kh = k_ref[b, :, h, :]
                vh = v_ref[b, :, h, :]
                s = lax.dot_general(
                    qh, kh, (((1,), (1,)), ((), ())),
                    preferred_element_type=jnp.float32) * 0.125
                s = jnp.where(mask, s, -1e9)
                m = jnp.max(s, axis=1, keepdims=True)
                w = jnp.exp(s - m)
                w = w / jnp.sum(w, axis=1, keepdims=True)
                ctxs.append(jnp.dot(w.astype(jnp.bfloat16), vh,
                                    preferred_element_type=jnp.float32))
            ctx = jnp.concatenate(ctxs, axis=1).astype(jnp.bfloat16)
            out_ref[b] = jnp.dot(ctx, wo_ref[...],
                                 preferred_element_type=jnp.float32)

        for r in range(LOG2_N):
            partner = my ^ (1 << r)
            send_buf[r] = out_ref[...]
            rdma = pltpu.make_async_remote_copy(
                src_ref=send_buf.at[r],
                dst_ref=recv_buf.at[r],
                send_sem=send_sems.at[r],
                recv_sem=recv_sems.at[r],
                device_id=(partner,),
                device_id_type=pl.DeviceIdType.MESH,
            )
            rdma.start()
            rdma.wait()
            out_ref[...] = out_ref[...] + recv_buf[r]

    return pl.pallas_call(
        body,
        out_shape=jax.ShapeDtypeStruct((B, SQ, D_MODEL), jnp.float32),
        in_specs=[pl.BlockSpec(memory_space=pltpu.VMEM)] * 5,
        out_specs=pl.BlockSpec(memory_space=pltpu.VMEM),
        scratch_shapes=[
            pltpu.VMEM((LOG2_N, B, SQ, D_MODEL), jnp.float32),
            pltpu.VMEM((LOG2_N, B, SQ, D_MODEL), jnp.float32),
            pltpu.SemaphoreType.DMA((LOG2_N,)),
            pltpu.SemaphoreType.DMA((LOG2_N,)),
        ],
    )(x_bf, wq_bf, k_bf, v_bf, wo_bf)


# device time: 49340 ns/iter; 1.7196x vs baseline; 1.7196x over previous
import jax
import jax.numpy as jnp
from jax import lax
from jax.experimental import pallas as pl
from jax.experimental.pallas import tpu as pltpu

N_DEV = 16
LOG2_N = 4
B, SQ, SKV, DH = 2, 256, 256, 64
H_LOC = 4
D_MODEL = 512
HD_LOC = H_LOC * DH

_LEN = [512 >> (r + 1) for r in range(LOG2_N)]


def kernel(x, Wq, K_ext, V_ext, Wo):
    pos = lax.axis_index("i")
    Wq_loc = lax.dynamic_slice(Wq, (0, pos * HD_LOC), (D_MODEL, HD_LOC))
    Wo_loc = lax.dynamic_slice(Wo, (pos * HD_LOC, 0), (HD_LOC, D_MODEL))

    x_bf = x.astype(jnp.bfloat16)
    wq_bf = Wq_loc.astype(jnp.bfloat16)
    k_bf = K_ext.astype(jnp.bfloat16)
    v_bf = V_ext.astype(jnp.bfloat16)
    wo_bf = Wo_loc.astype(jnp.bfloat16)

    def body(x_ref, wq_ref, k_ref, v_ref, wo_ref, out_ref,
             s0, s1, s2, s3, g0, g1, g2, g3, send_sems, recv_sems):
        my = lax.axis_index("i")
        send_stage = [s0, s1, s2, s3]
        recv_stage = [g0, g1, g2, g3]

        rb = lax.broadcasted_iota(jnp.int32, (SQ, SKV), 0) // 64
        cb = lax.broadcasted_iota(jnp.int32, (SQ, SKV), 1) // 64
        mask = (rb == cb) | ((cb % 4) == (rb % 4))

        for b in range(B):
            q = jnp.dot(x_ref[b], wq_ref[...],
                        preferred_element_type=jnp.float32)
            q = q.astype(jnp.bfloat16)
            ctxs = []
            for h in range(H_LOC):
                qh = q[:, h * DH:(h + 1) * DH]
                kh = k_ref[b, :, h, :]
                vh = v_ref[b, :, h, :]
                s = lax.dot_general(
                    qh, kh, (((1,), (1,)), ((), ())),
                    preferred_element_type=jnp.float32) * 0.125
                s = jnp.where(mask, s, -1e9)
                m = jnp.max(s, axis=1, keepdims=True)
                w = jnp.exp(s - m)
                w = w / jnp.sum(w, axis=1, keepdims=True)
                ctxs.append(jnp.dot(w.astype(jnp.bfloat16), vh,
                                    preferred_element_type=jnp.float32))
            ctx = jnp.concatenate(ctxs, axis=1).astype(jnp.bfloat16)
            out_ref[b] = jnp.dot(ctx, wo_ref[...],
                                 preferred_element_type=jnp.float32)

        def flat_slice(ref, lo, ln):
            return ref.at[lo // SQ, pl.ds(pl.multiple_of(lo % SQ, 32), ln)]

        lo = jnp.int32(0)
        for r in range(LOG2_N):
            ln = _LEN[r]
            partner = my ^ (1 << r)
            bit = (my >> r) & 1
            keep_lo = lo + bit * ln
            send_lo = lo + (1 - bit) * ln
            send_stage[r][...] = flat_slice(out_ref, send_lo, ln)[...].astype(
                jnp.bfloat16)
            rdma = pltpu.make_async_remote_copy(
                src_ref=send_stage[r],
                dst_ref=recv_stage[r],
                send_sem=send_sems.at[r],
                recv_sem=recv_sems.at[r],
                device_id=(partner,),
                device_id_type=pl.DeviceIdType.MESH,
            )
            rdma.start()
            rdma.wait()
            tgt = flat_slice(out_ref, keep_lo, ln)
            tgt[...] = tgt[...] + recv_stage[r][...].astype(jnp.float32)
            lo = keep_lo

        for r in reversed(range(LOG2_N)):
            ln = _LEN[r]
            partner = my ^ (1 << r)
            rdma = pltpu.make_async_remote_copy(
                src_ref=flat_slice(out_ref, lo, ln),
                dst_ref=flat_slice(out_ref, lo, ln),
                send_sem=send_sems.at[LOG2_N + r],
                recv_sem=recv_sems.at[LOG2_N + r],
                device_id=(partner,),
                device_id_type=pl.DeviceIdType.MESH,
            )
            rdma.start()
            rdma.wait()
            lo = lo & jnp.int32(~ln)

    return pl.pallas_call(
        body,
        out_shape=jax.ShapeDtypeStruct((B, SQ, D_MODEL), jnp.float32),
        in_specs=[pl.BlockSpec(memory_space=pltpu.VMEM)] * 5,
        out_specs=pl.BlockSpec(memory_space=pltpu.VMEM),
        scratch_shapes=[
            pltpu.VMEM((_LEN[0], D_MODEL), jnp.bfloat16),
            pltpu.VMEM((_LEN[1], D_MODEL), jnp.bfloat16),
            pltpu.VMEM((_LEN[2], D_MODEL), jnp.bfloat16),
            pltpu.VMEM((_LEN[3], D_MODEL), jnp.bfloat16),
            pltpu.VMEM((_LEN[0], D_MODEL), jnp.bfloat16),
            pltpu.VMEM((_LEN[1], D_MODEL), jnp.bfloat16),
            pltpu.VMEM((_LEN[2], D_MODEL), jnp.bfloat16),
            pltpu.VMEM((_LEN[3], D_MODEL), jnp.bfloat16),
            pltpu.SemaphoreType.DMA((2 * LOG2_N,)),
            pltpu.SemaphoreType.DMA((2 * LOG2_N,)),
        ],
    )(x_bf, wq_bf, k_bf, v_bf, wo_bf)


# device time: 34336 ns/iter; 2.4711x vs baseline; 1.4370x over previous
import jax
import jax.numpy as jnp
from jax import lax
from jax.experimental import pallas as pl
from jax.experimental.pallas import tpu as pltpu

N_DEV = 16
B, SQ, SKV, DH = 2, 256, 256, 64
H_LOC = 4
D_MODEL = 512
HD_LOC = H_LOC * DH
BLK = 512 // N_DEV


def kernel(x, Wq, K_ext, V_ext, Wo):
    pos = lax.axis_index("i")
    Wq_loc = lax.dynamic_slice(Wq, (0, pos * HD_LOC), (D_MODEL, HD_LOC))
    Wo_loc = lax.dynamic_slice(Wo, (pos * HD_LOC, 0), (HD_LOC, D_MODEL))

    def body(x_ref, wq_ref, k_ref, v_ref, wo_ref, out_ref,
             part_bf, blk_bf, p1_recv, part32,
             p1_send_sems, p1_recv_sems, p2_send_sems, p2_recv_sems):
        my = lax.axis_index("i")

        def p1_send(d):
            return pltpu.make_async_remote_copy(
                src_ref=part_bf.at[d // 8, pl.ds((d % 8) * BLK, BLK)],
                dst_ref=p1_recv.at[my],
                send_sem=p1_send_sems.at[d],
                recv_sem=p1_recv_sems.at[my],
                device_id=(jnp.int32(d),),
                device_id_type=pl.DeviceIdType.MESH,
            )

        p1_recv[my] = jnp.zeros((BLK, D_MODEL), jnp.bfloat16)

        rb = lax.broadcasted_iota(jnp.int32, (SQ, SKV), 0) // 64
        cb = lax.broadcasted_iota(jnp.int32, (SQ, SKV), 1) // 64
        mask = (rb == cb) | ((cb % 4) == (rb % 4))

        wq_b16 = wq_ref[...].astype(jnp.bfloat16)
        wo_b16 = wo_ref[...].astype(jnp.bfloat16)
        q_all = jnp.dot(x_ref[...].reshape(B * SQ, D_MODEL).astype(jnp.bfloat16),
                        wq_b16, preferred_element_type=jnp.float32)
        q_all = q_all.astype(jnp.bfloat16)
        for b in range(B):
            q = q_all[b * SQ:(b + 1) * SQ]
            ctxs = []
            for h in range(H_LOC):
                qh = q[:, h * DH:(h + 1) * DH]
                kh = k_ref[b, :, h, :].astype(jnp.bfloat16)
                vh = v_ref[b, :, h, :].astype(jnp.bfloat16)
                s = lax.dot_general(
                    qh, kh, (((1,), (1,)), ((), ())),
                    preferred_element_type=jnp.float32) * 0.125
                s = jnp.where(mask, s, -1e9)
                m = jnp.max(s, axis=1, keepdims=True)
                w = jnp.exp(s - m)
                w = w / jnp.sum(w, axis=1, keepdims=True)
                ctxs.append(jnp.dot(w.astype(jnp.bfloat16), vh,
                                    preferred_element_type=jnp.float32))
            ctx = jnp.concatenate(ctxs, axis=1).astype(jnp.bfloat16)
            part32[b] = jnp.dot(ctx, wo_b16,
                                preferred_element_type=jnp.float32)
            part_bf[b] = part32[b].astype(jnp.bfloat16)
            for d in range(b * 8, b * 8 + 8):
                pl.when(my != d)(lambda d=d: p1_send(d).start())

        for s in range(N_DEV):
            def _wait_p1(s=s):
                pltpu.make_async_remote_copy(
                    src_ref=part_bf.at[0, pl.ds(0, BLK)],
                    dst_ref=p1_recv.at[s],
                    send_sem=p1_send_sems.at[s],
                    recv_sem=p1_recv_sems.at[s],
                    device_id=(jnp.int32(s),),
                    device_id_type=pl.DeviceIdType.MESH,
                ).wait_recv()
            pl.when(my != s)(_wait_p1)

        def flat_slice(ref, d):
            return ref.at[d // 8, pl.ds(pl.multiple_of((d % 8) * BLK, BLK),
                                        BLK)]

        mine = flat_slice(part32, my)
        total = mine[...] + jnp.sum(p1_recv[...].astype(jnp.float32), axis=0)
        blk_bf[...] = total.astype(jnp.bfloat16)

        def p2_send(d):
            return pltpu.make_async_remote_copy(
                src_ref=blk_bf,
                dst_ref=flat_slice(out_ref, my),
                send_sem=p2_send_sems.at[d],
                recv_sem=p2_recv_sems.at[my],
                device_id=(jnp.int32(d),),
                device_id_type=pl.DeviceIdType.MESH,
            )

        flat_slice(out_ref, my)[...] = blk_bf[...]

        for d in range(N_DEV):
            pl.when(my != d)(lambda d=d: p2_send(d).start())

        for s in range(N_DEV):
            def _recv_p2(s=s):
                pltpu.make_async_remote_copy(
                    src_ref=blk_bf,
                    dst_ref=out_ref.at[s // 8, pl.ds((s % 8) * BLK, BLK)],
                    send_sem=p2_send_sems.at[s],
                    recv_sem=p2_recv_sems.at[s],
                    device_id=(jnp.int32(s),),
                    device_id_type=pl.DeviceIdType.MESH,
                ).wait_recv()
            pl.when(my != s)(_recv_p2)

        for d in range(N_DEV):
            pl.when(my != d)(lambda d=d: p1_send(d).wait_send())
            pl.when(my != d)(lambda d=d: p2_send(d).wait_send())

    return pl.pallas_call(
        body,
        out_shape=jax.ShapeDtypeStruct((B, SQ, D_MODEL), jnp.bfloat16),
        in_specs=[pl.BlockSpec(memory_space=pltpu.VMEM)] * 5,
        out_specs=pl.BlockSpec(memory_space=pltpu.VMEM),
        scratch_shapes=[
            pltpu.VMEM((B, SQ, D_MODEL), jnp.bfloat16),
            pltpu.VMEM((BLK, D_MODEL), jnp.bfloat16),
            pltpu.VMEM((N_DEV, BLK, D_MODEL), jnp.bfloat16),
            pltpu.VMEM((B, SQ, D_MODEL), jnp.float32),
            pltpu.SemaphoreType.DMA((N_DEV,)),
            pltpu.SemaphoreType.DMA((N_DEV,)),
            pltpu.SemaphoreType.DMA((N_DEV,)),
            pltpu.SemaphoreType.DMA((N_DEV,)),
        ],
    )(x, Wq_loc, K_ext, V_ext, Wo_loc)


# device time: 31991 ns/iter; 2.6522x vs baseline; 1.0733x over previous
import jax
import jax.numpy as jnp
from jax import lax
from jax.experimental import pallas as pl
from jax.experimental.pallas import tpu as pltpu

N_DEV = 16
B, SQ, SKV, DH = 2, 256, 256, 64
H_LOC = 4
D_MODEL = 512
HD_LOC = H_LOC * DH
BLK = 512 // N_DEV


def kernel(x, Wq, K_ext, V_ext, Wo):
    pos = lax.axis_index("i")
    Wq_loc = lax.dynamic_slice(Wq, (0, pos * HD_LOC), (D_MODEL, HD_LOC)) * 0.125
    Wo_loc = lax.dynamic_slice(Wo, (pos * HD_LOC, 0), (HD_LOC, D_MODEL))

    def body(x_ref, wq_ref, k_ref, v_ref, wo_ref, out_ref,
             part_bf, blk_bf, p1_recv, part32,
             p1_send_sems, p1_recv_sems, p2_send_sems, p2_recv_sems):
        my = lax.axis_index("i")

        def p1_send(d):
            return pltpu.make_async_remote_copy(
                src_ref=part_bf.at[d // 8, pl.ds((d % 8) * BLK, BLK)],
                dst_ref=p1_recv.at[my],
                send_sem=p1_send_sems.at[d],
                recv_sem=p1_recv_sems.at[my],
                device_id=(jnp.int32(d),),
                device_id_type=pl.DeviceIdType.MESH,
            )

        p1_recv[my] = jnp.zeros((BLK, D_MODEL), jnp.bfloat16)

        rb = lax.broadcasted_iota(jnp.int32, (SQ, SKV), 0) // 64
        cb = lax.broadcasted_iota(jnp.int32, (SQ, SKV), 1) // 64
        mask = (rb == cb) | ((cb % 4) == (rb % 4))

        wq_b16 = wq_ref[...].astype(jnp.bfloat16)
        wo_b16 = wo_ref[...].astype(jnp.bfloat16)
        k_b16 = k_ref[...].astype(jnp.bfloat16)
        v_b16 = v_ref[...].astype(jnp.bfloat16)
        q_all = jnp.dot(x_ref[...].reshape(B * SQ, D_MODEL).astype(jnp.bfloat16),
                        wq_b16, preferred_element_type=jnp.float32)
        q_all = q_all.astype(jnp.bfloat16)
        for b in range(B):
            q = q_all[b * SQ:(b + 1) * SQ]
            ctxs = []
            for h in range(H_LOC):
                qh = q[:, h * DH:(h + 1) * DH]
                kh = k_b16[b, :, h, :]
                vh = v_b16[b, :, h, :]
                s = lax.dot_general(
                    qh, kh, (((1,), (1,)), ((), ())),
                    preferred_element_type=jnp.float32)
                w = jnp.where(mask, jnp.exp(s), 0.0)
                denom = jnp.sum(w, axis=1, keepdims=True)
                cu = jnp.dot(w.astype(jnp.bfloat16), vh,
                             preferred_element_type=jnp.float32)
                ctxs.append(cu / denom)
            ctx = jnp.concatenate(ctxs, axis=1).astype(jnp.bfloat16)
            part32[b] = jnp.dot(ctx, wo_b16,
                                preferred_element_type=jnp.float32)
            part_bf[b] = part32[b].astype(jnp.bfloat16)
            for d in range(b * 8, b * 8 + 8):
                pl.when(my != d)(lambda d=d: p1_send(d).start())

        for s in range(N_DEV):
            def _wait_p1(s=s):
                pltpu.make_async_remote_copy(
                    src_ref=part_bf.at[0, pl.ds(0, BLK)],
                    dst_ref=p1_recv.at[s],
                    send_sem=p1_send_sems.at[s],
                    recv_sem=p1_recv_sems.at[s],
                    device_id=(jnp.int32(s),),
                    device_id_type=pl.DeviceIdType.MESH,
                ).wait_recv()
            pl.when(my != s)(_wait_p1)

        def flat_slice(ref, d):
            return ref.at[d // 8, pl.ds(pl.multiple_of((d % 8) * BLK, BLK),
                                        BLK)]

        mine = flat_slice(part32, my)
        total = mine[...] + jnp.sum(p1_recv[...].astype(jnp.float32), axis=0)
        blk_bf[...] = total.astype(jnp.bfloat16)

        def p2_send(d):
            return pltpu.make_async_remote_copy(
                src_ref=blk_bf,
                dst_ref=flat_slice(out_ref, my),
                send_sem=p2_send_sems.at[d],
                recv_sem=p2_recv_sems.at[my],
                device_id=(jnp.int32(d),),
                device_id_type=pl.DeviceIdType.MESH,
            )

        flat_slice(out_ref, my)[...] = blk_bf[...]

        for d in range(N_DEV):
            pl.when(my != d)(lambda d=d: p2_send(d).start())

        for s in range(N_DEV):
            def _recv_p2(s=s):
                pltpu.make_async_remote_copy(
                    src_ref=blk_bf,
                    dst_ref=out_ref.at[s // 8, pl.ds((s % 8) * BLK, BLK)],
                    send_sem=p2_send_sems.at[s],
                    recv_sem=p2_recv_sems.at[s],
                    device_id=(jnp.int32(s),),
                    device_id_type=pl.DeviceIdType.MESH,
                ).wait_recv()
            pl.when(my != s)(_recv_p2)

        for d in range(N_DEV):
            pl.when(my != d)(lambda d=d: p1_send(d).wait_send())
            pl.when(my != d)(lambda d=d: p2_send(d).wait_send())

    return pl.pallas_call(
        body,
        out_shape=jax.ShapeDtypeStruct((B, SQ, D_MODEL), jnp.bfloat16),
        in_specs=[pl.BlockSpec(memory_space=pltpu.VMEM)] * 5,
        out_specs=pl.BlockSpec(memory_space=pltpu.VMEM),
        scratch_shapes=[
            pltpu.VMEM((B, SQ, D_MODEL), jnp.bfloat16),
            pltpu.VMEM((BLK, D_MODEL), jnp.bfloat16),
            pltpu.VMEM((N_DEV, BLK, D_MODEL), jnp.bfloat16),
            pltpu.VMEM((B, SQ, D_MODEL), jnp.float32),
            pltpu.SemaphoreType.DMA((N_DEV,)),
            pltpu.SemaphoreType.DMA((N_DEV,)),
            pltpu.SemaphoreType.DMA((N_DEV,)),
            pltpu.SemaphoreType.DMA((N_DEV,)),
        ],
    )(x, Wq_loc, K_ext, V_ext, Wo_loc)
